# baseline (speedup 1.0000x reference)
import functools
import math

import jax
import jax.numpy as jnp
from jax import lax
from jax.experimental import pallas as pl
from jax.experimental.pallas import tpu as pltpu
from jax.experimental.pallas import tpu_sc as plsc

_LANES = 16
_PAD = 128
_CH = 128


@functools.lru_cache(maxsize=None)
def _build(batch: int, hist: int, hist_p: int, vocab: int, d_model: int,
           scale: float):
    info = plsc.get_sparse_core_info()
    nc, ns = info.num_cores, info.num_subcores
    nw = nc * ns
    assert batch % nw == 0
    tr_per_worker = batch // nw
    n_idx = tr_per_worker * hist
    assert n_idx % _CH == 0
    n_chunks = n_idx // _CH
    nbuf = 4
    assert n_chunks % nbuf == 0 and n_chunks >= 2 * nbuf
    n_groups = n_chunks // nbuf
    d_vecs = d_model // _LANES
    pk = _CH * d_model // _PAD
    out_rows = batch * hist * d_model // _PAD

    mesh = plsc.VectorSubcoreMesh(core_axis_name="c", subcore_axis_name="s")

    @functools.partial(
        pl.kernel,
        out_type=jax.ShapeDtypeStruct((out_rows, _PAD), jnp.float32),
        mesh=mesh,
        scratch_types=[
            pltpu.VMEM((tr_per_worker, hist_p), jnp.int32),
            pltpu.VMEM((nbuf * _CH,), jnp.int32),
            pltpu.VMEM((nbuf, _CH, _PAD), jnp.float32),
            pltpu.VMEM((2, pk, _PAD), jnp.float32),
            [pltpu.SemaphoreType.DMA] * nbuf,
            [pltpu.SemaphoreType.DMA] * 2,
        ],
        compiler_params=pltpu.CompilerParams(
            use_tc_tiling_on_sc=True, needs_layout_passes=False
        ),
    )
    def gather_scale(t128_hbm, tok_hbm, out_hbm, idx_v, idx1d_v, rows_v,
                     packed_v, gsems, wsems):
        wid = lax.axis_index("s") * nc + lax.axis_index("c")
        base_tr = wid * tr_per_worker
        base_out = wid * (n_idx * d_model // _PAD)
        pltpu.sync_copy(tok_hbm.at[pl.ds(base_tr, tr_per_worker)], idx_v)

        def repack(w, slot):
            for i in range(_CH // _LANES):
                p = w * _CH + i * _LANES + lax.iota(jnp.int32, _LANES)
                r = p // hist
                c = p - r * hist
                v = plsc.load_gather(idx_v, [r, c])
                idx1d_v[pl.ds(slot * _CH + i * _LANES, _LANES)] = v

        def gather(b):
            return pltpu.make_async_copy(
                t128_hbm.at[idx1d_v.at[pl.ds(b * _CH, _CH)]],
                rows_v.at[b],
                gsems[b],
            )

        def write(g, bp):
            off = pl.multiple_of(base_out + g * pk, pk)
            return pltpu.make_async_copy(
                packed_v.at[bp], out_hbm.at[pl.ds(off, pk)], wsems[bp]
            )

        repack(0, 0)
        repack(1, 1)
        repack(2, 2)
        gather(0).start()
        gather(1).start()

        def group_body(g0, _):
            for b in range(nbuf):
                g = g0 * nbuf + b
                bp = b % 2

                @pl.when(g + 3 < n_chunks)
                def _():
                    repack(g + 3, (b + 3) % nbuf)

                @pl.when(g >= 2)
                def _():
                    write(g - 2, bp).wait()

                @pl.when(g + 2 < n_chunks)
                def _():
                    gather((b + 2) % nbuf).start()

                gather(b).wait()

                @plsc.parallel_loop(0, pk, unroll=4)
                def _(j):
                    for h in range(2):
                        for d in range(d_vecs):
                            src = rows_v[b, 2 * j + h, pl.ds(d * _LANES, _LANES)]
                            dst = pl.ds(h * d_model + d * _LANES, _LANES)
                            packed_v[bp, j, dst] = src * scale

                write(g, bp).start()
            return 0

        lax.fori_loop(0, n_groups, group_body, 0)
        write(n_chunks - 2, 0).wait()
        write(n_chunks - 1, 1).wait()

    return gather_scale


def kernel(token, lookup_table):
    batch, hist = token.shape
    vocab, d_model = lookup_table.shape
    scale = math.sqrt(d_model)
    hist_p = -(-hist // _PAD) * _PAD
    t128 = jnp.pad(lookup_table, ((0, 0), (0, _PAD - d_model)))
    tok_p = jnp.pad(token.astype(jnp.int32), ((0, 0), (0, hist_p - hist)))
    fn = _build(batch, hist, hist_p, vocab, d_model, scale)
    out = fn(t128, tok_p)
    return out.reshape(batch, hist, d_model)
